# baseline (speedup 1.0000x reference)
import functools

import jax
import jax.numpy as jnp
from jax import lax
from jax.experimental import pallas as pl
from jax.experimental.pallas import tpu as pltpu
from jax.experimental.pallas import tpu_sc as plsc

NUM_Q = 8
INPUT_DIM = 512
CODE_DIM = 256
KSIZE = 2048
T = 8192

BT = 256


def _tables_kernel(emb_ref, wout_ref, table_ref, esq_ref):
    emb_q = emb_ref[0]
    wout_q = wout_ref[0]
    table_ref[0] = jax.lax.dot_general(
        emb_q, wout_q, (((1,), (1,)), ((), ())),
        preferred_element_type=jnp.float32)
    esq_ref[...] = jnp.sum(emb_q * emb_q, axis=-1).reshape(1, 1, KSIZE)


def _make_tables(emb_qkc, Wout_qdc):
    return pl.pallas_call(
        _tables_kernel,
        grid=(NUM_Q,),
        in_specs=[
            pl.BlockSpec((1, KSIZE, CODE_DIM), lambda q: (q, 0, 0)),
            pl.BlockSpec((1, INPUT_DIM, CODE_DIM), lambda q: (q, 0, 0)),
        ],
        out_specs=(
            pl.BlockSpec((1, KSIZE, INPUT_DIM), lambda q: (q, 0, 0)),
            pl.BlockSpec((1, 1, KSIZE), lambda q: (q, 0, 0)),
        ),
        out_shape=(
            jax.ShapeDtypeStruct((NUM_Q, KSIZE, INPUT_DIM), jnp.float32),
            jax.ShapeDtypeStruct((NUM_Q, 1, KSIZE), jnp.float32),
        ),
    )(emb_qkc, Wout_qdc)


def _stage_kernel(res_ref, dec_ref, win_ref, emb2_ref, esq_ref,
                  resout_ref, idx_ref, *, subtract):
    if subtract:
        res = res_ref[...] - dec_ref[...]
        resout_ref[...] = res
    else:
        res = res_ref[...]
    xp = jax.lax.dot_general(
        res, win_ref[...], (((1,), (1,)), ((), ())),
        preferred_element_type=jnp.float32)
    x_sq = jnp.sum(xp * xp, axis=-1, keepdims=True)
    cross2 = jax.lax.dot_general(
        xp, emb2_ref[...], (((1,), (1,)), ((), ())),
        preferred_element_type=jnp.float32)
    dist = (x_sq - cross2) + esq_ref[...]

    di = jax.lax.bitcast_convert_type(dist, jnp.int32)
    base = jax.lax.bitcast_convert_type(x_sq, jnp.int32)
    iota = jax.lax.broadcasted_iota(jnp.int32, (BT, KSIZE), 1)
    key = (di - base) * KSIZE + iota
    minkey = jnp.min(key, axis=-1, keepdims=True)
    idx = jnp.bitwise_and(minkey, KSIZE - 1)
    idx_ref[...] = idx.reshape(1, BT)


def _run_stage(res, dec_prev, win_q, emb2_q, esq_q):
    subtract = dec_prev is not None
    grid = (T // BT,)
    tile = pl.BlockSpec((BT, INPUT_DIM), lambda i: (i, 0))
    wspec = pl.BlockSpec((CODE_DIM, INPUT_DIM), lambda i: (0, 0))
    espec = pl.BlockSpec((KSIZE, CODE_DIM), lambda i: (0, 0))
    sspec = pl.BlockSpec((1, KSIZE), lambda i: (0, 0))
    ispec = pl.BlockSpec((1, BT), lambda i: (0, i))
    if subtract:
        in_specs = [tile, tile, wspec, espec, sspec]
        out_specs = (tile, ispec)
        out_shape = (
            jax.ShapeDtypeStruct((T, INPUT_DIM), jnp.float32),
            jax.ShapeDtypeStruct((1, T), jnp.int32),
        )
        args = (res, dec_prev, win_q, emb2_q, esq_q)
    else:
        in_specs = [tile, wspec, espec, sspec]
        out_specs = (ispec,)
        out_shape = (jax.ShapeDtypeStruct((1, T), jnp.int32),)
        args = (res, win_q, emb2_q, esq_q)

    def body(*refs):
        if subtract:
            r, d, w, e, s, ro, io = refs
            _stage_kernel(r, d, w, e, s, ro, io, subtract=True)
        else:
            r, w, e, s, io = refs
            _stage_kernel(r, None, w, e, s, None, io, subtract=False)

    outs = pl.pallas_call(
        body, grid=grid, in_specs=in_specs, out_specs=out_specs,
        out_shape=out_shape)(*args)
    if subtract:
        return outs[0], outs[1]
    return res, outs[0]


def _final_kernel(x_ref, res_ref, dec_ref, out_ref):
    out_ref[...] = x_ref[...] - (res_ref[...] - dec_ref[...])


def _run_final(x_td, res, dec):
    tile = pl.BlockSpec((BT, INPUT_DIM), lambda i: (i, 0))
    return pl.pallas_call(
        _final_kernel, grid=(T // BT,),
        in_specs=[tile, tile, tile], out_specs=tile,
        out_shape=jax.ShapeDtypeStruct((T, INPUT_DIM), jnp.float32),
    )(x_td, res, dec)


_SC_CHUNK = 64


def _sc_gather(table, idx_t):
    info = plsc.get_sparse_core_info()
    nw = info.num_cores * info.num_subcores
    b_per_w = T // nw
    n_chunks = b_per_w // _SC_CHUNK
    mesh = plsc.VectorSubcoreMesh(core_axis_name="c", subcore_axis_name="s")

    @functools.partial(
        pl.kernel, mesh=mesh,
        out_type=jax.ShapeDtypeStruct((T, INPUT_DIM), jnp.float32),
        scratch_types=[
            pltpu.VMEM((_SC_CHUNK,), jnp.int32),
            pltpu.VMEM((_SC_CHUNK, INPUT_DIM), jnp.float32),
            pltpu.SemaphoreType.DMA,
        ],
    )
    def gather(table_hbm, idx_hbm, out_hbm, idx_v, rows_v, sem):
        wid = lax.axis_index("s") * info.num_cores + lax.axis_index("c")
        base = wid * b_per_w
        for j in range(n_chunks):
            off = base + j * _SC_CHUNK
            pltpu.sync_copy(idx_hbm.at[pl.ds(off, _SC_CHUNK)], idx_v)
            pltpu.async_copy(table_hbm.at[idx_v], rows_v, sem).wait()
            pltpu.sync_copy(rows_v, out_hbm.at[pl.ds(off, _SC_CHUNK)])

    return gather(table, idx_t)


def kernel(x_td, Win_qcd, Wout_qdc, emb_qkc):
    emb2 = emb_qkc * 2.0
    tables, esq = _make_tables(emb_qkc, Wout_qdc)

    res = x_td
    dec = None
    codes_rows = []
    for q in range(NUM_Q):
        res, idx_row = _run_stage(
            res, dec, Win_qcd[q], emb2[q], esq[q])
        codes_rows.append(idx_row)
        dec = _sc_gather(tables[q], idx_row.reshape(T))
    out_td = _run_final(x_td, res, dec)
    codes_qt = jnp.concatenate(codes_rows, axis=0)
    return out_td, codes_qt


# speedup vs baseline: 1.5115x; 1.5115x over previous
import jax
import jax.numpy as jnp
from jax.experimental import pallas as pl

NUM_Q = 8
INPUT_DIM = 512
CODE_DIM = 256
KSIZE = 2048
T = 8192

HT = 256
BT = 2 * HT


def _esq_kernel(emb2_ref, esq_ref):
    e2 = emb2_ref[...]
    esq_ref[...] = 0.25 * jnp.sum(e2 * e2, axis=-1)


def _rvq_kernel(x_ref, win_ref, wout_ref, emb2_ref, esq_ref,
                out_ref, codes_ref):
    iota = jax.lax.broadcasted_iota(jnp.int32, (HT, KSIZE), 1)
    res = [x_ref[0:HT], x_ref[HT:BT]]
    out = [jnp.zeros_like(r) for r in res]
    idx_rows = [[], []]

    def stage_front(h, q):
        xp = jax.lax.dot_general(
            res[h], win_ref[q], (((1,), (1,)), ((), ())),
            preferred_element_type=jnp.float32)
        x_sq = jnp.sum(xp * xp, axis=-1, keepdims=True)
        cross2 = jax.lax.dot_general(
            xp, emb2_ref[q], (((1,), (1,)), ((), ())),
            preferred_element_type=jnp.float32)
        dist = (x_sq - cross2) + esq_ref[q][None, :]
        di = jax.lax.bitcast_convert_type(dist, jnp.int32)
        base = jax.lax.bitcast_convert_type(x_sq, jnp.int32)
        key = (di - base) * KSIZE + iota
        minkey = jnp.min(key, axis=-1, keepdims=True)
        return jnp.bitwise_and(minkey, KSIZE - 1)

    def stage_back(h, q, idx):
        onehot = (iota == idx).astype(jnp.float32)
        quant2 = jax.lax.dot_general(
            onehot, emb2_ref[q], (((1,), (0,)), ((), ())),
            preferred_element_type=jnp.float32)
        dec = 0.5 * jax.lax.dot_general(
            quant2, wout_ref[q], (((1,), (1,)), ((), ())),
            preferred_element_type=jnp.float32)
        res[h] = res[h] - dec
        out[h] = out[h] + dec

    for q in range(NUM_Q):
        idx0 = stage_front(0, q)
        idx1 = stage_front(1, q)
        idx_rows[0].append(idx0)
        idx_rows[1].append(idx1)
        stage_back(0, q, idx0)
        stage_back(1, q, idx1)

    out_ref[0:HT] = out[0]
    out_ref[HT:BT] = out[1]
    codes_ref[...] = jnp.concatenate(
        [jnp.concatenate([a.reshape(1, HT), b.reshape(1, HT)], axis=1)
         for a, b in zip(idx_rows[0], idx_rows[1])], axis=0)


def kernel(x_td, Win_qcd, Wout_qdc, emb_qkc):
    emb2 = emb_qkc * 2.0
    esq_qk = pl.pallas_call(
        _esq_kernel,
        out_shape=jax.ShapeDtypeStruct((NUM_Q, KSIZE), jnp.float32),
    )(emb2)

    grid = (T // BT,)
    out_td, codes_qt = pl.pallas_call(
        _rvq_kernel,
        grid=grid,
        in_specs=[
            pl.BlockSpec((BT, INPUT_DIM), lambda i: (i, 0)),
            pl.BlockSpec((NUM_Q, CODE_DIM, INPUT_DIM), lambda i: (0, 0, 0)),
            pl.BlockSpec((NUM_Q, INPUT_DIM, CODE_DIM), lambda i: (0, 0, 0)),
            pl.BlockSpec((NUM_Q, KSIZE, CODE_DIM), lambda i: (0, 0, 0)),
            pl.BlockSpec((NUM_Q, KSIZE), lambda i: (0, 0)),
        ],
        out_specs=(
            pl.BlockSpec((BT, INPUT_DIM), lambda i: (i, 0)),
            pl.BlockSpec((NUM_Q, BT), lambda i: (0, i)),
        ),
        out_shape=(
            jax.ShapeDtypeStruct((T, INPUT_DIM), jnp.float32),
            jax.ShapeDtypeStruct((NUM_Q, T), jnp.int32),
        ),
    )(x_td, Win_qcd, Wout_qdc, emb2, esq_qk)
    return out_td, codes_qt
